# Initial kernel scaffold; baseline (speedup 1.0000x reference)
import jax
import jax.numpy as jnp
from jax.experimental import pallas as pl


def kernel(x, segment_label, token_table, pos_table, seg_table):
    raise NotImplementedError("write your pallas kernel here")


# speedup vs baseline: 1.2706x; 1.2706x over previous
import jax
import jax.numpy as jnp
from jax import lax
from jax.experimental import pallas as pl
from jax.experimental.pallas import tpu as pltpu
from jax.experimental.pallas import tpu_sc as plsc

_EMB = 64
_B = 1024
_L = 200

_NC = 2
_NS = 16
_NW = _NC * _NS
_N = _B * _L
_CHUNK = 128
_NROWS = _N // _CHUNK
_NCHUNK = _NROWS // _NW
_NB = 5
_NG = _NCHUNK // _NB


def _posseg_body(pos_ref, seg_ref, out_ref):
    seg = seg_ref[...]
    pos = pos_ref[...]
    out_ref[...] = seg[:, None, :] + pos[None, :, :]


def _posseg(pos_table, seg_table):
    out = pl.pallas_call(
        _posseg_body,
        out_shape=jax.ShapeDtypeStruct((3, _L, _EMB), jnp.float32),
    )(pos_table, seg_table)
    return out.reshape(3 * _L, _EMB)


def _sc_body(x_hbm, s_hbm, tt_hbm, ps_hbm, out_hbm,
             idx_v, psidx_v, buf_v, tok_sems, add_sems, wr_sems):
    w = lax.axis_index("s") * _NC + lax.axis_index("c")
    row0 = w * _NCHUNK

    pltpu.sync_copy(x_hbm.at[w], idx_v)
    pltpu.sync_copy(s_hbm.at[w], psidx_v)

    iota = lax.iota(jnp.int32, 16)

    def idx_body(r, carry):
        fb = (row0 + r) * _CHUNK
        for cg in range(8):
            s16 = psidx_v[r, pl.ds(cg * 16, 16)]
            l16 = lax.rem(fb + cg * 16 + iota, _L)
            psidx_v[r, pl.ds(cg * 16, 16)] = s16 * _L + l16
        return carry

    lax.fori_loop(0, _NCHUNK, idx_body, 0)

    def out_rows(c):
        return out_hbm.at[pl.ds((row0 + c) * _CHUNK, _CHUNK)]

    def grp(g, carry):
        for s in range(_NB):
            c = g * _NB + s

            @pl.when(g > 0)
            def _():
                pltpu.make_async_copy(
                    buf_v.at[s], out_rows(c - _NB), wr_sems.at[s]).wait()

            pltpu.async_copy(tt_hbm.at[idx_v.at[c]], buf_v.at[s],
                             tok_sems.at[s])
        for s in range(_NB):
            c = g * _NB + s
            pltpu.make_async_copy(tt_hbm.at[idx_v.at[c]], buf_v.at[s],
                                  tok_sems.at[s]).wait()
            pltpu.async_copy(ps_hbm.at[psidx_v.at[c]], buf_v.at[s],
                             add_sems.at[s], add=True)
        for s in range(_NB):
            c = g * _NB + s
            pltpu.make_async_copy(ps_hbm.at[psidx_v.at[c]], buf_v.at[s],
                                  add_sems.at[s]).wait()
            pltpu.async_copy(buf_v.at[s], out_rows(c), wr_sems.at[s])
        return carry

    lax.fori_loop(0, _NG, grp, 0)

    for s in range(_NB):
        c = (_NG - 1) * _NB + s
        pltpu.make_async_copy(buf_v.at[s], out_rows(c), wr_sems.at[s]).wait()


def _sc_call(xf, sf, token_table, posseg):
    mesh = plsc.VectorSubcoreMesh(core_axis_name="c", subcore_axis_name="s")
    fn = pl.kernel(
        _sc_body,
        out_type=jax.ShapeDtypeStruct((_N, _EMB), jnp.float32),
        mesh=mesh,
        compiler_params=pltpu.CompilerParams(use_tc_tiling_on_sc=False),
        scratch_types=[
            pltpu.VMEM((_NCHUNK, _CHUNK), jnp.int32),
            pltpu.VMEM((_NCHUNK, _CHUNK), jnp.int32),
            pltpu.VMEM((_NB, _CHUNK, _EMB), jnp.float32),
            pltpu.SemaphoreType.DMA((_NB,)),
            pltpu.SemaphoreType.DMA((_NB,)),
            pltpu.SemaphoreType.DMA((_NB,)),
        ],
    )
    return fn(xf, sf, token_table, posseg)


def kernel(x, segment_label, token_table, pos_table, seg_table):
    posseg = _posseg(pos_table, seg_table)
    xf = x.reshape(_NW, _NCHUNK, _CHUNK).astype(jnp.int32)
    sf = segment_label.reshape(_NW, _NCHUNK, _CHUNK).astype(jnp.int32)
    out = _sc_call(xf, sf, token_table, posseg)
    return out.reshape(_B, _L, _EMB)
